# baseline (speedup 1.0000x reference)
import functools

import jax
import jax.numpy as jnp
from jax import lax
from jax.experimental import pallas as pl
from jax.experimental.pallas import tpu as pltpu
from jax.experimental.pallas import tpu_sc as plsc

_BATCH = 16384


@functools.partial(jax.jit, static_argnames=())
def _gather_sc(states, V):
    info = plsc.get_sparse_core_info()
    num_cores = 1
    nw = num_cores * info.num_subcores
    b_per_w = _BATCH // nw
    n_chunks = 3
    chunk = -(-b_per_w // n_chunks) // 8 * 8
    sizes = [chunk] * (n_chunks - 1) + [b_per_w - (n_chunks - 1) * chunk]
    offs = [i * chunk for i in range(n_chunks)]
    mesh = plsc.VectorSubcoreMesh(
        core_axis_name="c", subcore_axis_name="s", num_cores=num_cores)

    @functools.partial(
        pl.kernel,
        mesh=mesh,
        out_type=jax.ShapeDtypeStruct((_BATCH,), jnp.float32),
        scratch_types=[
            pltpu.VMEM((b_per_w,), jnp.int32),
            pltpu.VMEM((b_per_w,), jnp.float32),
        ] + [pltpu.SemaphoreType.DMA] * n_chunks,
    )
    def body(states_hbm, table_hbm, out_hbm, idx_v, vals_v, *sems):
        wid = lax.axis_index("s") * num_cores + lax.axis_index("c")
        base = wid * b_per_w
        stages = [
            pltpu.async_copy(
                states_hbm.at[pl.ds(base + offs[i], sizes[i])],
                idx_v.at[pl.ds(offs[i], sizes[i])], sems[i])
            for i in range(n_chunks)
        ]
        gathers = []
        for i in range(n_chunks):
            stages[i].wait()
            gathers.append(pltpu.async_copy(
                table_hbm.at[idx_v.at[pl.ds(offs[i], sizes[i])]],
                vals_v.at[pl.ds(offs[i], sizes[i])], sems[i]))
        outs = []
        for i in range(n_chunks):
            gathers[i].wait()
            outs.append(pltpu.async_copy(
                vals_v.at[pl.ds(offs[i], sizes[i])],
                out_hbm.at[pl.ds(base + offs[i], sizes[i])], sems[i]))
        for o in outs:
            o.wait()

    return body(states, V)


def kernel(states, V):
    return _gather_sc(states.astype(jnp.int32), V)
